# Initial kernel scaffold; baseline (speedup 1.0000x reference)
import functools

import jax
import jax.numpy as jnp
from jax import lax
from jax.experimental import pallas as pl
from jax.experimental.pallas import tpu as pltpu
from jax.experimental.pallas import tpu_sc as plsc

N = 10000
E = 320000
EL = 100000
D = 128
R_DEC = 2.0
T_DEC = 1.0

NC = 2
NS = 16
NW = NC * NS
EPT = E // NW
EPT_H = E // NS
CH = 80
ROWS_PT = N // NS

DCH = 80
NCHD = EL // DCH
MAXK = (NCHD + NW - 1) // NW

_SC_MESH = plsc.VectorSubcoreMesh(core_axis_name="c", subcore_axis_name="s")


def _hist_body(dst_hbm, zeros_hbm, deg_hbm, acc, idx_v, ones_v, sem):
    del sem
    cid = lax.axis_index("c")
    sid = lax.axis_index("s")

    @pl.when(cid == 0)
    def _():
        ones = jnp.full((16,), 1.0, dtype=jnp.float32)
        for j in range(CH // 16):
            ones_v[pl.ds(j * 16, 16)] = ones

        @pl.when(sid == 0)
        def _():
            pltpu.sync_copy(zeros_hbm, acc)

        plsc.subcore_barrier()

        def chunk(k, carry):
            base = sid * EPT_H + k * CH
            pltpu.sync_copy(dst_hbm.at[pl.ds(base, CH)], idx_v)
            pltpu.sync_copy(ones_v, acc.at[idx_v], add=True)
            return carry

        lax.fori_loop(0, EPT_H // CH, chunk, 0)
        plsc.subcore_barrier()

        @pl.when(sid == 0)
        def _():
            pltpu.sync_copy(acc, deg_hbm)


_hist_kernel = pl.kernel(
    _hist_body,
    out_type=jax.ShapeDtypeStruct((N,), jnp.float32),
    mesh=_SC_MESH,
    scratch_types=[
        pltpu.VMEM_SHARED((N,), jnp.float32),
        pltpu.VMEM((CH,), jnp.int32),
        pltpu.VMEM((CH,), jnp.float32),
        pltpu.SemaphoreType.DMA,
    ],
)


def _agg_body(y_hbm, src_hbm, dst_hbm, zeros_hbm, outa_hbm, outb_hbm,
              acc, sidx, didx, rows, sem):
    cid = lax.axis_index("c")
    sid = lax.axis_index("s")
    wid = cid * NS + sid
    rbase = sid * ROWS_PT

    @pl.when(cid == 0)
    def _():
        pltpu.sync_copy(y_hbm.at[pl.ds(rbase, ROWS_PT)],
                        acc.at[pl.ds(rbase, ROWS_PT)])

    @pl.when(cid == 1)
    def _():
        pltpu.sync_copy(zeros_hbm.at[pl.ds(rbase, ROWS_PT)],
                        acc.at[pl.ds(rbase, ROWS_PT)])

    plsc.subcore_barrier()

    def chunk(k, carry):
        base = wid * EPT + k * CH
        pltpu.sync_copy(src_hbm.at[pl.ds(base, CH)], sidx)
        pltpu.sync_copy(dst_hbm.at[pl.ds(base, CH)], didx)
        pltpu.async_copy(y_hbm.at[sidx], rows, sem).wait()
        pltpu.sync_copy(rows, acc.at[didx], add=True)
        return carry

    lax.fori_loop(0, EPT // CH, chunk, 0)
    plsc.subcore_barrier()

    @pl.when(cid == 0)
    def _():
        pltpu.sync_copy(acc.at[pl.ds(rbase, ROWS_PT)],
                        outa_hbm.at[pl.ds(rbase, ROWS_PT)])

    @pl.when(cid == 1)
    def _():
        pltpu.sync_copy(acc.at[pl.ds(rbase, ROWS_PT)],
                        outb_hbm.at[pl.ds(rbase, ROWS_PT)])


_agg_kernel = pl.kernel(
    _agg_body,
    out_type=(jax.ShapeDtypeStruct((N, D), jnp.float32),
              jax.ShapeDtypeStruct((N, D), jnp.float32)),
    mesh=_SC_MESH,
    scratch_types=[
        pltpu.VMEM_SHARED((N, D), jnp.float32),
        pltpu.VMEM((CH,), jnp.int32),
        pltpu.VMEM((CH,), jnp.int32),
        pltpu.VMEM((CH, D), jnp.float32),
        pltpu.SemaphoreType.DMA,
    ],
)


def _dec_body(h_hbm, ein_hbm, eout_hbm, out_hbm,
              ia, ib, ra, rb, pv, sema, semb):
    cid = lax.axis_index("c")
    sid = lax.axis_index("s")
    wid = cid * NS + sid
    iota = lax.iota(jnp.int32, 16)

    def chunk(k, carry):
        c = wid + NW * k

        @pl.when(c < NCHD)
        def _():
            base = c * DCH
            pltpu.sync_copy(ein_hbm.at[pl.ds(base, DCH)], ia)
            pltpu.sync_copy(eout_hbm.at[pl.ds(base, DCH)], ib)
            cp_a = pltpu.async_copy(h_hbm.at[ia], ra, sema)
            cp_b = pltpu.async_copy(h_hbm.at[ib], rb, semb)
            cp_a.wait()
            cp_b.wait()
            for g in range(DCH // 16):
                rowi = g * 16 + iota

                def fstep(f, sq_acc):
                    colf = iota * 0 + f
                    a = plsc.load_gather(ra, [rowi, colf])
                    b = plsc.load_gather(rb, [rowi, colf])
                    d = a - b
                    return sq_acc + d * d

                sq = lax.fori_loop(0, D, fstep,
                                   jnp.zeros((16,), jnp.float32))
                pv[pl.ds(g * 16, 16)] = (
                    1.0 / (jnp.exp((sq + 1e-12 - R_DEC) / T_DEC) + 1.0))
            pltpu.sync_copy(pv, out_hbm.at[pl.ds(base, DCH)])

        return carry

    lax.fori_loop(0, MAXK, chunk, 0)


_dec_kernel = pl.kernel(
    _dec_body,
    out_type=jax.ShapeDtypeStruct((EL,), jnp.float32),
    mesh=_SC_MESH,
    scratch_types=[
        pltpu.VMEM((DCH,), jnp.int32),
        pltpu.VMEM((DCH,), jnp.int32),
        pltpu.VMEM((DCH, D), jnp.float32),
        pltpu.VMEM((DCH, D), jnp.float32),
        pltpu.VMEM((DCH,), jnp.float32),
        pltpu.SemaphoreType.DMA,
        pltpu.SemaphoreType.DMA,
    ],
)


BLK = 1000


def _tc1_call(x, W1, deg1):
    def body(x_ref, w_ref, deg_ref, y_ref):
        dinv = lax.rsqrt(deg_ref[...] + 1.0)
        xw = jnp.dot(x_ref[...], w_ref[...],
                     preferred_element_type=jnp.float32)
        y_ref[...] = xw * dinv

    return pl.pallas_call(
        body,
        grid=(N // BLK,),
        in_specs=[pl.BlockSpec((BLK, D), lambda i: (i, 0)),
                  pl.BlockSpec((D, D), lambda i: (0, 0)),
                  pl.BlockSpec((BLK, 1), lambda i: (i, 0))],
        out_specs=pl.BlockSpec((BLK, D), lambda i: (i, 0)),
        out_shape=jax.ShapeDtypeStruct((N, D), jnp.float32),
    )(x, W1, deg1)


def _tc2_call(a0, a1, deg1, b1, W2):
    def body(a0_ref, a1_ref, deg_ref, b_ref, w_ref, y_ref):
        dinv = lax.rsqrt(deg_ref[...] + 1.0)
        h = jnp.maximum(dinv * (a0_ref[...] + a1_ref[...]) + b_ref[...], 0.0)
        y_ref[...] = jnp.dot(h, w_ref[...],
                             preferred_element_type=jnp.float32) * dinv

    return pl.pallas_call(
        body,
        grid=(N // BLK,),
        in_specs=[pl.BlockSpec((BLK, D), lambda i: (i, 0)),
                  pl.BlockSpec((BLK, D), lambda i: (i, 0)),
                  pl.BlockSpec((BLK, 1), lambda i: (i, 0)),
                  pl.BlockSpec((D,), lambda i: (0,)),
                  pl.BlockSpec((D, D), lambda i: (0, 0))],
        out_specs=pl.BlockSpec((BLK, D), lambda i: (i, 0)),
        out_shape=jax.ShapeDtypeStruct((N, D), jnp.float32),
    )(a0, a1, deg1, b1, W2)


def _tc3_call(c0, c1, deg1, b2):
    def body(c0_ref, c1_ref, deg_ref, b_ref, h_ref):
        dinv = lax.rsqrt(deg_ref[...] + 1.0)
        h_ref[...] = dinv * (c0_ref[...] + c1_ref[...]) + b_ref[...]

    return pl.pallas_call(
        body,
        grid=(N // BLK,),
        in_specs=[pl.BlockSpec((BLK, D), lambda i: (i, 0)),
                  pl.BlockSpec((BLK, D), lambda i: (i, 0)),
                  pl.BlockSpec((BLK, 1), lambda i: (i, 0)),
                  pl.BlockSpec((D,), lambda i: (0,))],
        out_specs=pl.BlockSpec((BLK, D), lambda i: (i, 0)),
        out_shape=jax.ShapeDtypeStruct((N, D), jnp.float32),
    )(c0, c1, deg1, b2)


def kernel(node_features, edge_index, edge_label_index, W1, b1, W2, b2):
    x = node_features.astype(jnp.float32)
    src = edge_index[0]
    dst = edge_index[1]
    ein = edge_label_index[0]
    eout = edge_label_index[1]
    zeros_nd = jnp.zeros((N, D), jnp.float32)
    zeros_n = jnp.zeros((N,), jnp.float32)

    deg = _hist_kernel(dst, zeros_n)
    deg1 = deg.reshape(N, 1)

    y1 = _tc1_call(x, W1, deg1)
    a0, a1 = _agg_kernel(y1, src, dst, zeros_nd)
    y2 = _tc2_call(a0, a1, deg1, b1, W2)
    c0, c1 = _agg_kernel(y2, src, dst, zeros_nd)
    h2 = _tc3_call(c0, c1, deg1, b2)

    return _dec_kernel(h2, ein, eout)


# speedup vs baseline: 9.9001x; 9.9001x over previous
import functools

import jax
import jax.numpy as jnp
from jax import lax
from jax.experimental import pallas as pl
from jax.experimental.pallas import tpu as pltpu
from jax.experimental.pallas import tpu_sc as plsc

N = 10000
E = 320000
EL = 100000
D = 128
R_DEC = 2.0
T_DEC = 1.0

NC = 2
NS = 16
NW = NC * NS
EPT = E // NW
EPT_H = E // NS
CH = 80
ROWS_PT = 624

DCH = 80
NCHD = EL // DCH
MAXK = (NCHD + NW - 1) // NW

_SC_MESH = plsc.VectorSubcoreMesh(core_axis_name="c", subcore_axis_name="s")


def _hist_body(dst_hbm, zeros_hbm, deg_hbm, acc, idx_v, ones_v, sem):
    del sem
    cid = lax.axis_index("c")
    sid = lax.axis_index("s")

    @pl.when(cid == 0)
    def _():
        ones = jnp.full((16,), 1.0, dtype=jnp.float32)
        for j in range(CH // 16):
            ones_v[pl.ds(j * 16, 16)] = ones

        @pl.when(sid == 0)
        def _():
            pltpu.sync_copy(zeros_hbm, acc)

        plsc.subcore_barrier()

        def chunk(k, carry):
            base = sid * EPT_H + k * CH
            pltpu.sync_copy(dst_hbm.at[pl.ds(base, CH)], idx_v)
            pltpu.sync_copy(ones_v, acc.at[idx_v], add=True)
            return carry

        lax.fori_loop(0, EPT_H // CH, chunk, 0)
        plsc.subcore_barrier()

        @pl.when(sid == 0)
        def _():
            pltpu.sync_copy(acc, deg_hbm)


_hist_kernel = pl.kernel(
    _hist_body,
    out_type=jax.ShapeDtypeStruct((N,), jnp.float32),
    mesh=_SC_MESH,
    scratch_types=[
        pltpu.VMEM_SHARED((N,), jnp.float32),
        pltpu.VMEM((CH,), jnp.int32),
        pltpu.VMEM((CH,), jnp.float32),
        pltpu.SemaphoreType.DMA,
    ],
)


def _row_split(sid, fn):
    @pl.when(sid < NS - 1)
    def _():
        fn(sid * ROWS_PT, ROWS_PT)

    @pl.when(sid == NS - 1)
    def _():
        fn((NS - 1) * ROWS_PT, N - (NS - 1) * ROWS_PT)


def _agg_body(y_hbm, src_hbm, dst_hbm, zeros_hbm, outa_hbm, outb_hbm,
              acc, sidx, didx, rows, sem):
    cid = lax.axis_index("c")
    sid = lax.axis_index("s")
    wid = cid * NS + sid

    def init(base, cnt):
        @pl.when(cid == 0)
        def _():
            pltpu.sync_copy(y_hbm.at[pl.ds(base, cnt)],
                            acc.at[pl.ds(base, cnt)])

        @pl.when(cid == 1)
        def _():
            pltpu.sync_copy(zeros_hbm.at[pl.ds(base, cnt)],
                            acc.at[pl.ds(base, cnt)])

    _row_split(sid, init)
    plsc.subcore_barrier()

    def chunk(k, carry):
        base = wid * EPT + k * CH
        pltpu.sync_copy(src_hbm.at[pl.ds(base, CH)], sidx)
        pltpu.sync_copy(dst_hbm.at[pl.ds(base, CH)], didx)
        pltpu.async_copy(y_hbm.at[sidx], rows, sem).wait()
        pltpu.sync_copy(rows, acc.at[didx], add=True)
        return carry

    lax.fori_loop(0, EPT // CH, chunk, 0)
    plsc.subcore_barrier()

    def flush(base, cnt):
        @pl.when(cid == 0)
        def _():
            pltpu.sync_copy(acc.at[pl.ds(base, cnt)],
                            outa_hbm.at[pl.ds(base, cnt)])

        @pl.when(cid == 1)
        def _():
            pltpu.sync_copy(acc.at[pl.ds(base, cnt)],
                            outb_hbm.at[pl.ds(base, cnt)])

    _row_split(sid, flush)


_agg_kernel = pl.kernel(
    _agg_body,
    out_type=(jax.ShapeDtypeStruct((N, D), jnp.float32),
              jax.ShapeDtypeStruct((N, D), jnp.float32)),
    mesh=_SC_MESH,
    scratch_types=[
        pltpu.VMEM_SHARED((N, D), jnp.float32),
        pltpu.VMEM((CH,), jnp.int32),
        pltpu.VMEM((CH,), jnp.int32),
        pltpu.VMEM((CH, D), jnp.float32),
        pltpu.SemaphoreType.DMA,
    ],
)


def _dec_body(h_hbm, ein_hbm, eout_hbm, embi_hbm, embo_hbm,
              ia, ib, ra, rb, sema, semb):
    cid = lax.axis_index("c")
    sid = lax.axis_index("s")
    wid = cid * NS + sid

    def chunk(k, carry):
        c = wid + NW * k

        @pl.when(c < NCHD)
        def _():
            base = c * DCH
            pltpu.sync_copy(ein_hbm.at[pl.ds(base, DCH)], ia)
            pltpu.sync_copy(eout_hbm.at[pl.ds(base, DCH)], ib)
            cp_a = pltpu.async_copy(h_hbm.at[ia], ra, sema)
            cp_b = pltpu.async_copy(h_hbm.at[ib], rb, semb)
            cp_a.wait()
            cp_b.wait()
            pltpu.sync_copy(ra, embi_hbm.at[pl.ds(base, DCH)])
            pltpu.sync_copy(rb, embo_hbm.at[pl.ds(base, DCH)])

        return carry

    lax.fori_loop(0, MAXK, chunk, 0)


_dec_kernel = pl.kernel(
    _dec_body,
    out_type=(jax.ShapeDtypeStruct((EL, D), jnp.float32),
              jax.ShapeDtypeStruct((EL, D), jnp.float32)),
    mesh=_SC_MESH,
    scratch_types=[
        pltpu.VMEM((DCH,), jnp.int32),
        pltpu.VMEM((DCH,), jnp.int32),
        pltpu.VMEM((DCH, D), jnp.float32),
        pltpu.VMEM((DCH, D), jnp.float32),
        pltpu.SemaphoreType.DMA,
        pltpu.SemaphoreType.DMA,
    ],
)


BLK = 1000


def _tc1_call(x, W1, deg1):
    def body(x_ref, w_ref, deg_ref, y_ref):
        dinv = lax.rsqrt(deg_ref[...] + 1.0)
        xw = jnp.dot(x_ref[...], w_ref[...],
                     preferred_element_type=jnp.float32)
        y_ref[...] = xw * dinv

    return pl.pallas_call(
        body,
        grid=(N // BLK,),
        in_specs=[pl.BlockSpec((BLK, D), lambda i: (i, 0)),
                  pl.BlockSpec((D, D), lambda i: (0, 0)),
                  pl.BlockSpec((BLK, 1), lambda i: (i, 0))],
        out_specs=pl.BlockSpec((BLK, D), lambda i: (i, 0)),
        out_shape=jax.ShapeDtypeStruct((N, D), jnp.float32),
    )(x, W1, deg1)


def _tc2_call(a0, a1, deg1, b1, W2):
    def body(a0_ref, a1_ref, deg_ref, b_ref, w_ref, y_ref):
        dinv = lax.rsqrt(deg_ref[...] + 1.0)
        h = jnp.maximum(dinv * (a0_ref[...] + a1_ref[...]) + b_ref[...], 0.0)
        y_ref[...] = jnp.dot(h, w_ref[...],
                             preferred_element_type=jnp.float32) * dinv

    return pl.pallas_call(
        body,
        grid=(N // BLK,),
        in_specs=[pl.BlockSpec((BLK, D), lambda i: (i, 0)),
                  pl.BlockSpec((BLK, D), lambda i: (i, 0)),
                  pl.BlockSpec((BLK, 1), lambda i: (i, 0)),
                  pl.BlockSpec((D,), lambda i: (0,)),
                  pl.BlockSpec((D, D), lambda i: (0, 0))],
        out_specs=pl.BlockSpec((BLK, D), lambda i: (i, 0)),
        out_shape=jax.ShapeDtypeStruct((N, D), jnp.float32),
    )(a0, a1, deg1, b1, W2)


def _tc3_call(c0, c1, deg1, b2):
    def body(c0_ref, c1_ref, deg_ref, b_ref, h_ref):
        dinv = lax.rsqrt(deg_ref[...] + 1.0)
        h_ref[...] = dinv * (c0_ref[...] + c1_ref[...]) + b_ref[...]

    return pl.pallas_call(
        body,
        grid=(N // BLK,),
        in_specs=[pl.BlockSpec((BLK, D), lambda i: (i, 0)),
                  pl.BlockSpec((BLK, D), lambda i: (i, 0)),
                  pl.BlockSpec((BLK, 1), lambda i: (i, 0)),
                  pl.BlockSpec((D,), lambda i: (0,))],
        out_specs=pl.BlockSpec((BLK, D), lambda i: (i, 0)),
        out_shape=jax.ShapeDtypeStruct((N, D), jnp.float32),
    )(c0, c1, deg1, b2)


DBLK = 2000


def _tc4_call(embi, embo):
    def body(a_ref, b_ref, p_ref):
        d = a_ref[...] - b_ref[...]
        sq = jnp.sum(d * d, axis=1, keepdims=True)
        p_ref[...] = 1.0 / (jnp.exp((sq + 1e-12 - R_DEC) / T_DEC) + 1.0)

    return pl.pallas_call(
        body,
        grid=(EL // DBLK,),
        in_specs=[pl.BlockSpec((DBLK, D), lambda i: (i, 0)),
                  pl.BlockSpec((DBLK, D), lambda i: (i, 0))],
        out_specs=pl.BlockSpec((DBLK, 1), lambda i: (i, 0)),
        out_shape=jax.ShapeDtypeStruct((EL, 1), jnp.float32),
    )(embi, embo)


def kernel(node_features, edge_index, edge_label_index, W1, b1, W2, b2):
    x = node_features.astype(jnp.float32)
    src = edge_index[0]
    dst = edge_index[1]
    ein = edge_label_index[0]
    eout = edge_label_index[1]
    zeros_nd = jnp.zeros((N, D), jnp.float32)
    zeros_n = jnp.zeros((N,), jnp.float32)

    deg = _hist_kernel(dst, zeros_n)
    deg1 = deg.reshape(N, 1)

    y1 = _tc1_call(x, W1, deg1)
    a0, a1 = _agg_kernel(y1, src, dst, zeros_nd)
    y2 = _tc2_call(a0, a1, deg1, b1, W2)
    c0, c1 = _agg_kernel(y2, src, dst, zeros_nd)
    h2 = _tc3_call(c0, c1, deg1, b2)

    embi, embo = _dec_kernel(h2, ein, eout)
    return _tc4_call(embi, embo).reshape(EL)
